# baseline (speedup 1.0000x reference)
import functools

import jax
import jax.numpy as jnp
import numpy as np
from jax import lax
from jax.experimental import pallas as pl
from jax.experimental.pallas import tpu as pltpu
from jax.experimental.pallas import tpu_sc as plsc

NUM_BASIS = 16
GRID_SIZE = 1000
Z_MIN = -8.0
Z_MAX = 8.0
GRID_PAD = 1024
TABLE_PAD = 1040

U_SCALE = np.float32((GRID_SIZE - 1) / (Z_MAX - Z_MIN + 1e-8))


def _make_I_grid_T_flat():
    z_grid = np.linspace(Z_MIN, Z_MAX, GRID_SIZE)
    knots = np.linspace(Z_MIN, Z_MAX, NUM_BASIS)
    d = np.abs(z_grid[:, None] - knots[None, :])
    dx = (Z_MAX - Z_MIN) / (NUM_BASIS - 1)
    H = np.clip(1.0 - d / dx, 0.0, None)
    H = H / (H.sum(axis=1, keepdims=True) + 1e-08)
    dz = z_grid[1] - z_grid[0]
    I = np.cumsum(H * dz, axis=0)
    I_max = I[-1, :].copy()
    I_max[I_max <= 0] = 1.0
    I = I / I_max[None, :]
    I_T = np.ascontiguousarray(I.T.astype(np.float32))
    out = np.zeros((NUM_BASIS, GRID_PAD), np.float32)
    out[:, :GRID_SIZE] = I_T
    return out.reshape(-1)


_IG_FLAT = _make_I_grid_T_flat()


def _make_sc_kernel(n_rows, n_cols):
    info = plsc.get_sparse_core_info()
    nc, ns, nl = info.num_cores, info.num_subcores, info.num_lanes
    nw = nc * ns
    rows_per_w = n_rows // nw
    crows = 16
    nchunk = rows_per_w // crows
    vecs_per_row = n_cols // nl
    nblk = GRID_PAD // nl
    mesh = plsc.VectorSubcoreMesh(core_axis_name="c", subcore_axis_name="s")

    @functools.partial(
        pl.kernel,
        mesh=mesh,
        out_type=jax.ShapeDtypeStruct((n_rows, n_cols), jnp.float32),
        scratch_types=[
            pltpu.VMEM((NUM_BASIS * GRID_PAD,), jnp.float32),
            pltpu.VMEM((nl,), jnp.float32),
            pltpu.VMEM((nl,), jnp.float32),
            pltpu.VMEM((2 * nl,), jnp.float32),
            pltpu.VMEM((TABLE_PAD,), jnp.float32),
            pltpu.VMEM((GRID_PAD,), jnp.float32),
            pltpu.VMEM((crows, n_cols), jnp.float32),
            pltpu.VMEM((crows, n_cols), jnp.float32),
            pltpu.VMEM((crows, n_cols), jnp.float32),
            pltpu.VMEM((crows, n_cols), jnp.float32),
            pltpu.SemaphoreType.DMA,
            pltpu.SemaphoreType.DMA,
            pltpu.SemaphoreType.DMA,
            pltpu.SemaphoreType.DMA,
        ],
        compiler_params=pltpu.CompilerParams(needs_layout_passes=False),
    )
    def sck(ig_hbm, params_hbm, rw_hbm, z_hbm, out_hbm,
            ig_v, par_v, w_v, w2_v, t0_v, t1_v,
            zb0, zb1, ob0, ob1, is0, is1, os0, os1):
        wid = lax.axis_index("s") * nc + lax.axis_index("c")
        base = wid * rows_per_w
        zbufs, obufs = (zb0, zb1), (ob0, ob1)
        isems, osems = (is0, is1), (os0, os1)

        pltpu.sync_copy(params_hbm, par_v)
        pltpu.sync_copy(rw_hbm, w_v)
        pltpu.sync_copy(ig_hbm, ig_v)

        x = w_v[...]
        q = jnp.exp(-jnp.abs(x))
        a1 = 1.0 + q
        L = (2.0 * q) / (2.0 + q)
        L = L + a1 * jnp.exp(-L) - 1.0
        L = L + a1 * jnp.exp(-L) - 1.0
        L = L + a1 * jnp.exp(-L) - 1.0
        w2_v[pl.ds(nl, nl)] = jnp.maximum(x, 0.0) + L

        def splat(ref, j):
            return plsc.load_gather(ref, [jnp.full((nl,), j, jnp.int32)])

        an = splat(par_v, 1)
        bn = splat(par_v, 2)
        cn = splat(par_v, 3)
        wspl = [splat(w2_v, nl + m) for m in range(NUM_BASIS)]

        @plsc.parallel_loop(0, nblk, unroll=2)
        def _tbl(b):
            acc = wspl[0] * ig_v[pl.ds(b * nl, nl)]
            for m in range(1, NUM_BASIS):
                acc = acc + wspl[m] * ig_v[pl.ds(m * GRID_PAD + b * nl, nl)]
            t0_v[pl.ds(b * nl, nl)] = cn + bn * acc

        t0_v[pl.ds(GRID_PAD, TABLE_PAD - GRID_PAD)] = jnp.zeros(
            (TABLE_PAD - GRID_PAD,), jnp.float32)

        @plsc.parallel_loop(0, nblk, unroll=2)
        def _shift(b):
            idxv = lax.iota(jnp.int32, nl) + (b * nl + 1)
            t1_v[pl.ds(b * nl, nl)] = plsc.load_gather(t0_v, [idxv])

        def compute(zb, ob):
            @plsc.parallel_loop(0, crows * vecs_per_row, unroll=16)
            def _body(i):
                r = i // vecs_per_row
                col = (i % vecs_per_row) * nl
                zv = zb[r, pl.ds(col, nl)]
                zc = jnp.minimum(jnp.maximum(zv, Z_MIN), Z_MAX)
                u = (zc - Z_MIN) * U_SCALE
                idx = u.astype(jnp.int32)
                fr = u - idx.astype(jnp.float32)
                t0 = plsc.load_gather(t0_v, [idx])
                t1 = plsc.load_gather(t1_v, [idx])
                gneg = an * zc + (t0 + fr * (t1 - t0))
                ob[r, pl.ds(col, nl)] = 1.0 / (1.0 + jnp.exp(gneg))

        in_h = [None, None]
        out_h = [None, None]
        in_h[0] = pltpu.async_copy(
            z_hbm.at[pl.ds(base, crows), :], zb0, is0)
        for c in range(nchunk):
            b = c & 1
            r0 = base + c * crows
            in_h[b].wait()
            if c + 1 < nchunk:
                nb = (c + 1) & 1
                in_h[nb] = pltpu.async_copy(
                    z_hbm.at[pl.ds(base + (c + 1) * crows, crows), :],
                    zbufs[nb], isems[nb])
            if c >= 2:
                out_h[b].wait()
            compute(zbufs[b], obufs[b])
            out_h[b] = pltpu.async_copy(
                obufs[b], out_hbm.at[pl.ds(r0, crows), :], osems[b])
        out_h[(nchunk - 2) & 1].wait()
        out_h[(nchunk - 1) & 1].wait()

    return sck


def kernel(z, raw_weights, alpha, beta, bias):
    orig_shape = z.shape
    n_total = int(np.prod(orig_shape))
    n_cols = orig_shape[-1]
    n_rows = n_total // n_cols
    params = jnp.concatenate([
        jnp.zeros((1,), jnp.float32),
        (-alpha).reshape(1), (-beta).reshape(1), (-bias).reshape(1),
        jnp.zeros((12,), jnp.float32)]).astype(jnp.float32)
    sck = _make_sc_kernel(n_rows, n_cols)
    out = sck(jnp.asarray(_IG_FLAT), params, raw_weights.astype(jnp.float32),
              z.reshape(n_rows, n_cols))
    return out.reshape(orig_shape)


# speedup vs baseline: 1.1255x; 1.1255x over previous
import functools

import jax
import jax.numpy as jnp
import numpy as np
from jax import lax
from jax.experimental import pallas as pl
from jax.experimental.pallas import tpu as pltpu
from jax.experimental.pallas import tpu_sc as plsc

NUM_BASIS = 16
GRID_SIZE = 1000
Z_MIN = -8.0
Z_MAX = 8.0
GRID_PAD = 1024
TABLE_PAD = 1040

U_SCALE = np.float32((GRID_SIZE - 1) / (Z_MAX - Z_MIN + 1e-8))


def _make_I_grid_T_flat():
    z_grid = np.linspace(Z_MIN, Z_MAX, GRID_SIZE)
    knots = np.linspace(Z_MIN, Z_MAX, NUM_BASIS)
    d = np.abs(z_grid[:, None] - knots[None, :])
    dx = (Z_MAX - Z_MIN) / (NUM_BASIS - 1)
    H = np.clip(1.0 - d / dx, 0.0, None)
    H = H / (H.sum(axis=1, keepdims=True) + 1e-08)
    dz = z_grid[1] - z_grid[0]
    I = np.cumsum(H * dz, axis=0)
    I_max = I[-1, :].copy()
    I_max[I_max <= 0] = 1.0
    I = I / I_max[None, :]
    I_T = np.ascontiguousarray(I.T.astype(np.float32))
    out = np.zeros((NUM_BASIS, GRID_PAD), np.float32)
    out[:, :GRID_SIZE] = I_T
    return out.reshape(-1)


_IG_FLAT = _make_I_grid_T_flat()


def _make_sc_kernel(n_rows, n_cols):
    info = plsc.get_sparse_core_info()
    nc, ns, nl = info.num_cores, info.num_subcores, info.num_lanes
    nw = nc * ns
    rows_per_w = n_rows // nw
    crows = 16
    nchunk = rows_per_w // crows
    vecs_per_row = n_cols // nl
    nblk = GRID_PAD // nl
    mesh = plsc.VectorSubcoreMesh(core_axis_name="c", subcore_axis_name="s")

    @functools.partial(
        pl.kernel,
        mesh=mesh,
        out_type=jax.ShapeDtypeStruct((n_rows, n_cols), jnp.float32),
        scratch_types=[
            pltpu.VMEM((NUM_BASIS * GRID_PAD,), jnp.float32),
            pltpu.VMEM((nl,), jnp.float32),
            pltpu.VMEM((nl,), jnp.float32),
            pltpu.VMEM((2 * nl,), jnp.float32),
            pltpu.VMEM((TABLE_PAD,), jnp.float32),
            pltpu.VMEM((GRID_PAD,), jnp.float32),
            pltpu.VMEM((crows, n_cols), jnp.float32),
            pltpu.VMEM((crows, n_cols), jnp.float32),
            pltpu.VMEM((crows, n_cols), jnp.float32),
            pltpu.VMEM((crows, n_cols), jnp.float32),
            pltpu.SemaphoreType.DMA,
            pltpu.SemaphoreType.DMA,
            pltpu.SemaphoreType.DMA,
            pltpu.SemaphoreType.DMA,
        ],
        compiler_params=pltpu.CompilerParams(needs_layout_passes=False),
    )
    def sck(ig_hbm, params_hbm, rw_hbm, z_hbm, out_hbm,
            ig_v, par_v, w_v, w2_v, t0_v, t1_v,
            zb0, zb1, ob0, ob1, is0, is1, os0, os1):
        wid = lax.axis_index("s") * nc + lax.axis_index("c")
        base = wid * rows_per_w
        zbufs, obufs = (zb0, zb1), (ob0, ob1)
        isems, osems = (is0, is1), (os0, os1)

        in_h = [None, None]
        out_h = [None, None]
        in_h[0] = pltpu.async_copy(
            z_hbm.at[pl.ds(base, crows), :], zb0, is0)

        pltpu.sync_copy(params_hbm, par_v)
        pltpu.sync_copy(rw_hbm, w_v)
        pltpu.sync_copy(ig_hbm, ig_v)

        x = w_v[...]
        q = jnp.exp(-jnp.abs(x))
        a1 = 1.0 + q
        L = (2.0 * q) / (2.0 + q)
        L = L + a1 * jnp.exp(-L) - 1.0
        L = L + a1 * jnp.exp(-L) - 1.0
        L = L + a1 * jnp.exp(-L) - 1.0
        w2_v[pl.ds(nl, nl)] = jnp.maximum(x, 0.0) + L

        def splat(ref, j):
            return plsc.load_gather(ref, [jnp.full((nl,), j, jnp.int32)])

        an = splat(par_v, 1)
        bn = splat(par_v, 2)
        cn = splat(par_v, 3)
        wspl = [splat(w2_v, nl + m) for m in range(NUM_BASIS)]

        @plsc.parallel_loop(0, nblk, unroll=2)
        def _tbl(b):
            acc = wspl[0] * ig_v[pl.ds(b * nl, nl)]
            for m in range(1, NUM_BASIS):
                acc = acc + wspl[m] * ig_v[pl.ds(m * GRID_PAD + b * nl, nl)]
            t0_v[pl.ds(b * nl, nl)] = cn + bn * acc

        t0_v[pl.ds(GRID_PAD, TABLE_PAD - GRID_PAD)] = jnp.zeros(
            (TABLE_PAD - GRID_PAD,), jnp.float32)

        @plsc.parallel_loop(0, nblk, unroll=2)
        def _shift(b):
            idxv = lax.iota(jnp.int32, nl) + (b * nl + 1)
            t1_v[pl.ds(b * nl, nl)] = plsc.load_gather(t0_v, [idxv])

        def compute(zb, ob):
            @plsc.parallel_loop(0, crows * vecs_per_row, unroll=8)
            def _body(i):
                r = i // vecs_per_row
                col = (i % vecs_per_row) * nl
                zv = zb[r, pl.ds(col, nl)]
                zc = jnp.minimum(jnp.maximum(zv, Z_MIN), Z_MAX)
                u = (zc - Z_MIN) * U_SCALE
                idx = u.astype(jnp.int32)
                fr = u - idx.astype(jnp.float32)
                t0 = plsc.load_gather(t0_v, [idx])
                t1 = plsc.load_gather(t1_v, [idx])
                gneg = an * zc + (t0 + fr * (t1 - t0))
                ob[r, pl.ds(col, nl)] = 1.0 / (1.0 + jnp.exp(gneg))

        for c in range(nchunk):
            b = c & 1
            r0 = base + c * crows
            in_h[b].wait()
            if c + 1 < nchunk:
                nb = (c + 1) & 1
                in_h[nb] = pltpu.async_copy(
                    z_hbm.at[pl.ds(base + (c + 1) * crows, crows), :],
                    zbufs[nb], isems[nb])
            if c >= 2:
                out_h[b].wait()
            compute(zbufs[b], obufs[b])
            out_h[b] = pltpu.async_copy(
                obufs[b], out_hbm.at[pl.ds(r0, crows), :], osems[b])
        out_h[(nchunk - 2) & 1].wait()
        out_h[(nchunk - 1) & 1].wait()

    return sck


def kernel(z, raw_weights, alpha, beta, bias):
    orig_shape = z.shape
    n_total = int(np.prod(orig_shape))
    n_cols = orig_shape[-1]
    n_rows = n_total // n_cols
    params = jnp.concatenate([
        jnp.zeros((1,), jnp.float32),
        (-alpha).reshape(1), (-beta).reshape(1), (-bias).reshape(1),
        jnp.zeros((12,), jnp.float32)]).astype(jnp.float32)
    sck = _make_sc_kernel(n_rows, n_cols)
    out = sck(jnp.asarray(_IG_FLAT), params, raw_weights.astype(jnp.float32),
              z.reshape(n_rows, n_cols))
    return out.reshape(orig_shape)
